# baseline (speedup 1.0000x reference)
import jax
import jax.numpy as jnp
from jax import lax
from jax.experimental import pallas as pl
from jax.experimental.pallas import tpu as pltpu
from jax.experimental.pallas import tpu_sc as plsc

_H = 32
_NC = 2
_NS = 16
_LANES = 16


def _sc_tile_kernel(w_hbm, tile_hbm, w_v, sem):
    n = _H * _H
    per_w = n // (_NC * _NS)
    wid = lax.axis_index("s") * _NC + lax.axis_index("c")
    base = wid * per_w
    pltpu.sync_copy(w_hbm, w_v)
    c = _H // 2
    copies = []
    for t in range(per_w):
        ij = base + t
        i = ij // _H
        j = ij % _H
        z = jnp.maximum(jnp.abs(c - j) + jnp.abs(c - i) - 1, 0)
        copies.append(
            pltpu.make_async_copy(
                w_v.at[pl.ds(z, 1)], tile_hbm.at[pl.ds(ij, 1)], sem
            )
        )
    for cp in copies:
        cp.start()
    for cp in copies:
        cp.wait()


def _fanout_kernel(t_ref, o_ref, tile_ref, sems):
    tile_ref[0, :, :] = t_ref[...]
    tile_ref[1, :, :] = t_ref[...]
    nb = o_ref.shape[0]
    copies = [
        pltpu.make_async_copy(
            tile_ref, o_ref.at[pl.ds(2 * q, 2)], sems.at[q]
        )
        for q in range(nb // 2)
    ]
    for c in copies:
        c.start()
    for c in copies:
        c.wait()


def kernel(x, W):
    b = x.shape[0]
    nv, nf = W.shape
    h, w = x.shape[-2], x.shape[-1]
    n = h * w
    per_w = n // (_NC * _NS)
    mesh = plsc.VectorSubcoreMesh(core_axis_name="c", subcore_axis_name="s")
    tile = pl.kernel(
        _sc_tile_kernel,
        out_type=jax.ShapeDtypeStruct((n, nf), jnp.float32),
        mesh=mesh,
        scratch_types=[
            pltpu.MemorySpace.VMEM((nv, nf), jnp.float32),
            pltpu.SemaphoreType.DMA,
        ],
    )(W)
    out = pl.pallas_call(
        _fanout_kernel,
        in_specs=[pl.BlockSpec(memory_space=pltpu.MemorySpace.VMEM)],
        out_specs=pl.BlockSpec(memory_space=pltpu.MemorySpace.HBM),
        out_shape=jax.ShapeDtypeStruct((b, n, nf), jnp.float32),
        scratch_shapes=[
            pltpu.MemorySpace.VMEM((2, n, nf), jnp.float32),
            pltpu.SemaphoreType.DMA((b // 2,)),
        ],
    )(tile)
    return out.reshape(b, h, w, nf).transpose(0, 3, 1, 2)


# speedup vs baseline: 3.2333x; 3.2333x over previous
import jax
import jax.numpy as jnp
from jax.experimental import pallas as pl
from jax.experimental.pallas import tpu as pltpu


def _pos_kernel(w_ref, o_ref, tile_ref, sems):
    h = w = 32
    cy, cx = h // 2, w // 2
    n = h * w
    nrows = w_ref.shape[0]
    ij = jax.lax.broadcasted_iota(jnp.int32, (n, nrows), 0)
    i = ij // w
    j = ij % w
    z = jnp.maximum(jnp.abs(cx - j) + jnp.abs(cy - i) - 1, 0)
    cols = jax.lax.broadcasted_iota(jnp.int32, (n, nrows), 1)
    onehot = (cols == z).astype(jnp.float32)
    tile = jnp.dot(
        onehot, w_ref[...], preferred_element_type=jnp.float32
    )
    ncopies = tile_ref.shape[0]
    for t in range(ncopies):
        tile_ref[t, :, :] = tile
    nb = o_ref.shape[0]
    copies = [
        pltpu.make_async_copy(
            tile_ref, o_ref.at[pl.ds(ncopies * q, ncopies)], sems.at[q]
        )
        for q in range(nb // ncopies)
    ]
    for c in copies:
        c.start()
    for c in copies:
        c.wait()


def kernel(x, W):
    b = x.shape[0]
    nf = W.shape[1]
    h, w = x.shape[-2], x.shape[-1]
    n = h * w
    out = pl.pallas_call(
        _pos_kernel,
        in_specs=[pl.BlockSpec(memory_space=pltpu.MemorySpace.VMEM)],
        out_specs=pl.BlockSpec(memory_space=pltpu.MemorySpace.HBM),
        out_shape=jax.ShapeDtypeStruct((b, n, nf), jnp.float32),
        scratch_shapes=[
            pltpu.MemorySpace.VMEM((4, n, nf), jnp.float32),
            pltpu.SemaphoreType.DMA((b // 4,)),
        ],
    )(W)
    return out.reshape(b, h, w, nf).transpose(0, 3, 1, 2)


# speedup vs baseline: 3.2693x; 1.0112x over previous
import jax
import jax.numpy as jnp
from jax.experimental import pallas as pl
from jax.experimental.pallas import tpu as pltpu


def _pos_kernel(w_ref, o_ref, tile_ref, sems):
    h = w = 32
    cy, cx = h // 2, w // 2
    n = h * w
    nrows = w_ref.shape[0]
    ij = jax.lax.broadcasted_iota(jnp.int32, (n, nrows), 0)
    i = ij // w
    j = ij % w
    z = jnp.maximum(jnp.abs(cx - j) + jnp.abs(cy - i) - 1, 0)
    cols = jax.lax.broadcasted_iota(jnp.int32, (n, nrows), 1)
    onehot = (cols == z).astype(jnp.float32)
    tile = jnp.dot(
        onehot, w_ref[...], preferred_element_type=jnp.float32
    )
    tile_ref[0, :, :] = tile
    tile_ref[1, :, :] = tile
    nb = o_ref.shape[0]
    copies = [
        pltpu.make_async_copy(
            tile_ref, o_ref.at[pl.ds(2 * q, 2)], sems.at[q]
        )
        for q in range(nb // 2)
    ]
    for c in copies:
        c.start()
    for c in copies:
        c.wait()


def kernel(x, W):
    b = x.shape[0]
    nf = W.shape[1]
    h, w = x.shape[-2], x.shape[-1]
    n = h * w
    out = pl.pallas_call(
        _pos_kernel,
        in_specs=[pl.BlockSpec(memory_space=pltpu.MemorySpace.VMEM)],
        out_specs=pl.BlockSpec(memory_space=pltpu.MemorySpace.HBM),
        out_shape=jax.ShapeDtypeStruct((b, n, nf), jnp.float32),
        scratch_shapes=[
            pltpu.MemorySpace.VMEM((2, n, nf), jnp.float32),
            pltpu.SemaphoreType.DMA((b // 2,)),
        ],
    )(W)
    return out.reshape(b, h, w, nf).transpose(0, 3, 1, 2)
